# Initial kernel scaffold; baseline (speedup 1.0000x reference)
import jax
import jax.numpy as jnp
from jax.experimental import pallas as pl


def kernel(x, h, q, ws_w, wk_w, wv_w, g_mem, g_h):
    raise NotImplementedError("write your pallas kernel here")


# speedup vs baseline: 4.2716x; 4.2716x over previous
import math

import jax
import jax.numpy as jnp
from jax import lax
from jax.experimental import pallas as pl
from jax.experimental.pallas import tpu as pltpu

_BB = 8


def _fused_kernel(x_ref, h_ref, qkt_ref, ws_ref, wv_ref, gmem_ref, gh_ref,
                  o_ref, *, eps):
    bb, nw, dsl = x_ref.shape
    md = ws_ref.shape[1]

    xb = x_ref[...].reshape(bb * nw, dsl).astype(jnp.bfloat16)

    s = jnp.dot(xb, ws_ref[...], preferred_element_type=jnp.float32)
    sb = s.astype(jnp.bfloat16)

    sv = jnp.dot(sb, wv_ref[...], preferred_element_type=jnp.float32)

    t = jnp.dot(sb, qkt_ref[...], preferred_element_type=jnp.float32)

    ms = t.shape[-1]
    t3 = t.reshape(bb, nw, ms)

    m = jnp.maximum(jnp.max(t3, axis=1, keepdims=True), 0.0)
    p = jnp.exp(t3 - m)
    denom = jnp.sum(p, axis=1, keepdims=True) + jnp.exp(-m)
    p = (p / denom).astype(jnp.bfloat16)

    sv3 = sv.astype(jnp.bfloat16).reshape(bb, nw, md)
    mem = lax.dot_general(p, sv3, (((1,), (1,)), ((0,), (0,))),
                          preferred_element_type=jnp.float32)

    ms1 = jnp.mean(mem * mem, axis=(1, 2), keepdims=True)
    hn = h_ref[...] + mem * lax.rsqrt(ms1 + eps) * gmem_ref[...]
    ms2 = jnp.mean(hn * hn, axis=(1, 2), keepdims=True)
    o_ref[...] = (hn * lax.rsqrt(ms2 + eps) * gh_ref[...]).astype(o_ref.dtype)


def kernel(x, h, q, ws_w, wk_w, wv_w, g_mem, g_h):
    B, seqlen, dim = x.shape
    _, memory_size, memory_dim = h.shape
    summary_len = ws_w.shape[1] // dim
    nw = seqlen // summary_len
    dsl = dim * summary_len
    eps = float(jnp.finfo(jnp.float32).eps)
    scale = 1.0 / math.sqrt(memory_dim)

    xr = x.reshape(B, nw, dsl)

    ws_t = ws_w.T.astype(jnp.bfloat16)
    wv_t = wv_w.T.astype(jnp.bfloat16)
    qkt = ((q @ wk_w) * scale).T.astype(jnp.bfloat16)

    bb = _BB
    return pl.pallas_call(
        lambda *refs: _fused_kernel(*refs, eps=eps),
        out_shape=jax.ShapeDtypeStruct((B, memory_size, memory_dim), h.dtype),
        grid=(B // bb,),
        in_specs=[
            pl.BlockSpec((bb, nw, dsl), lambda i: (i, 0, 0)),
            pl.BlockSpec((bb, memory_size, memory_dim), lambda i: (i, 0, 0)),
            pl.BlockSpec((memory_dim, memory_size), lambda i: (0, 0)),
            pl.BlockSpec((dsl, memory_dim), lambda i: (0, 0)),
            pl.BlockSpec((memory_dim, memory_dim), lambda i: (0, 0)),
            pl.BlockSpec((memory_size, memory_dim), lambda i: (0, 0)),
            pl.BlockSpec((memory_size, memory_dim), lambda i: (0, 0)),
        ],
        out_specs=pl.BlockSpec((bb, memory_size, memory_dim),
                               lambda i: (i, 0, 0)),
        compiler_params=pltpu.CompilerParams(
            dimension_semantics=("parallel",),
            vmem_limit_bytes=64 * 1024 * 1024,
        ),
    )(xr, h, qkt, ws_t, wv_t, g_mem, g_h)


# speedup vs baseline: 4.8055x; 1.1250x over previous
import math

import jax
import jax.numpy as jnp
from jax import lax
from jax.experimental import pallas as pl
from jax.experimental.pallas import tpu as pltpu

_BB = 16


def _fused_kernel(x_ref, h_ref, qkt_ref, ws_ref, wv_ref, gmem_ref, gh_ref,
                  o_ref, *, eps):
    bb, nw, dsl = x_ref.shape
    md = ws_ref.shape[1]

    xb = x_ref[...].reshape(bb * nw, dsl).astype(jnp.bfloat16)

    s = jnp.dot(xb, ws_ref[...], preferred_element_type=jnp.float32)
    sb = s.astype(jnp.bfloat16)

    sv = jnp.dot(sb, wv_ref[...], preferred_element_type=jnp.float32)

    t = jnp.dot(sb, qkt_ref[...], preferred_element_type=jnp.float32)

    ms = t.shape[-1]
    t3 = t.reshape(bb, nw, ms)

    m = jnp.maximum(jnp.max(t3, axis=1, keepdims=True), 0.0)
    p = jnp.exp(t3 - m)
    denom = jnp.sum(p, axis=1, keepdims=True) + jnp.exp(-m)
    p = (p / denom).astype(jnp.bfloat16)

    sv3 = sv.astype(jnp.bfloat16).reshape(bb, nw, md)
    mem = lax.dot_general(p, sv3, (((1,), (1,)), ((0,), (0,))),
                          preferred_element_type=jnp.float32)

    ms1 = jnp.mean(mem * mem, axis=(1, 2), keepdims=True)
    hn = h_ref[...] + mem * lax.rsqrt(ms1 + eps) * gmem_ref[...]
    ms2 = jnp.mean(hn * hn, axis=(1, 2), keepdims=True)
    o_ref[...] = (hn * lax.rsqrt(ms2 + eps) * gh_ref[...]).astype(o_ref.dtype)


def kernel(x, h, q, ws_w, wk_w, wv_w, g_mem, g_h):
    B, seqlen, dim = x.shape
    _, memory_size, memory_dim = h.shape
    summary_len = ws_w.shape[1] // dim
    nw = seqlen // summary_len
    dsl = dim * summary_len
    eps = float(jnp.finfo(jnp.float32).eps)
    scale = 1.0 / math.sqrt(memory_dim)

    xr = x.reshape(B, nw, dsl)

    ws_t = ws_w.T.astype(jnp.bfloat16)
    wv_t = wv_w.T.astype(jnp.bfloat16)
    qkt = ((q @ wk_w) * scale).T.astype(jnp.bfloat16)

    bb = _BB
    return pl.pallas_call(
        lambda *refs: _fused_kernel(*refs, eps=eps),
        out_shape=jax.ShapeDtypeStruct((B, memory_size, memory_dim), h.dtype),
        grid=(B // bb,),
        in_specs=[
            pl.BlockSpec((bb, nw, dsl), lambda i: (i, 0, 0)),
            pl.BlockSpec((bb, memory_size, memory_dim), lambda i: (i, 0, 0)),
            pl.BlockSpec((memory_dim, memory_size), lambda i: (0, 0)),
            pl.BlockSpec((dsl, memory_dim), lambda i: (0, 0)),
            pl.BlockSpec((memory_dim, memory_dim), lambda i: (0, 0)),
            pl.BlockSpec((memory_size, memory_dim), lambda i: (0, 0)),
            pl.BlockSpec((memory_size, memory_dim), lambda i: (0, 0)),
        ],
        out_specs=pl.BlockSpec((bb, memory_size, memory_dim),
                               lambda i: (i, 0, 0)),
        compiler_params=pltpu.CompilerParams(
            dimension_semantics=("parallel",),
            vmem_limit_bytes=64 * 1024 * 1024,
        ),
    )(xr, h, qkt, ws_t, wv_t, g_mem, g_h)


# speedup vs baseline: 5.0545x; 1.0518x over previous
import math

import jax
import jax.numpy as jnp
from jax import lax
from jax.experimental import pallas as pl
from jax.experimental.pallas import tpu as pltpu

_BB = 32


def _fused_kernel(x_ref, h_ref, qkt_ref, ws_ref, wv_ref, gmem_ref, gh_ref,
                  o_ref, *, eps):
    bb, nw, dsl = x_ref.shape
    md = ws_ref.shape[1]

    xb = x_ref[...].reshape(bb * nw, dsl).astype(jnp.bfloat16)

    s = jnp.dot(xb, ws_ref[...], preferred_element_type=jnp.float32)
    sb = s.astype(jnp.bfloat16)

    sv = jnp.dot(sb, wv_ref[...], preferred_element_type=jnp.float32)

    t = jnp.dot(sb, qkt_ref[...], preferred_element_type=jnp.float32)

    ms = t.shape[-1]
    t3 = t.reshape(bb, nw, ms)

    m = jnp.maximum(jnp.max(t3, axis=1, keepdims=True), 0.0)
    p = jnp.exp(t3 - m)
    denom = jnp.sum(p, axis=1, keepdims=True) + jnp.exp(-m)
    p = (p / denom).astype(jnp.bfloat16)

    sv3 = sv.astype(jnp.bfloat16).reshape(bb, nw, md)
    mem = lax.dot_general(p, sv3, (((1,), (1,)), ((0,), (0,))),
                          preferred_element_type=jnp.float32)

    ms1 = jnp.mean(mem * mem, axis=(1, 2), keepdims=True)
    hn = h_ref[...] + mem * lax.rsqrt(ms1 + eps) * gmem_ref[...]
    ms2 = jnp.mean(hn * hn, axis=(1, 2), keepdims=True)
    o_ref[...] = (hn * lax.rsqrt(ms2 + eps) * gh_ref[...]).astype(o_ref.dtype)


def kernel(x, h, q, ws_w, wk_w, wv_w, g_mem, g_h):
    B, seqlen, dim = x.shape
    _, memory_size, memory_dim = h.shape
    summary_len = ws_w.shape[1] // dim
    nw = seqlen // summary_len
    dsl = dim * summary_len
    eps = float(jnp.finfo(jnp.float32).eps)
    scale = 1.0 / math.sqrt(memory_dim)

    xr = x.reshape(B, nw, dsl)

    ws_t = ws_w.T.astype(jnp.bfloat16)
    wv_t = wv_w.T.astype(jnp.bfloat16)
    qkt = ((q @ wk_w) * scale).T.astype(jnp.bfloat16)

    bb = _BB
    return pl.pallas_call(
        lambda *refs: _fused_kernel(*refs, eps=eps),
        out_shape=jax.ShapeDtypeStruct((B, memory_size, memory_dim), h.dtype),
        grid=(B // bb,),
        in_specs=[
            pl.BlockSpec((bb, nw, dsl), lambda i: (i, 0, 0)),
            pl.BlockSpec((bb, memory_size, memory_dim), lambda i: (i, 0, 0)),
            pl.BlockSpec((memory_dim, memory_size), lambda i: (0, 0)),
            pl.BlockSpec((dsl, memory_dim), lambda i: (0, 0)),
            pl.BlockSpec((memory_dim, memory_dim), lambda i: (0, 0)),
            pl.BlockSpec((memory_size, memory_dim), lambda i: (0, 0)),
            pl.BlockSpec((memory_size, memory_dim), lambda i: (0, 0)),
        ],
        out_specs=pl.BlockSpec((bb, memory_size, memory_dim),
                               lambda i: (i, 0, 0)),
        compiler_params=pltpu.CompilerParams(
            dimension_semantics=("parallel",),
            vmem_limit_bytes=64 * 1024 * 1024,
        ),
    )(xr, h, qkt, ws_t, wv_t, g_mem, g_h)


# speedup vs baseline: 12.9930x; 2.5706x over previous
import math

import jax
import jax.numpy as jnp
from jax import lax
from jax.experimental import pallas as pl
from jax.experimental.pallas import tpu as pltpu

_BB = 32


def _fused_kernel(x_ref, h_ref, qkt_ref, ws_ref, wv_ref, gmem_ref, gh_ref,
                  o_ref, *, eps):
    bb, seqlen, dim = x_ref.shape
    dsl = ws_ref.shape[0]
    nw = (bb * seqlen * dim) // (bb * dsl)
    md = ws_ref.shape[1]

    xb = x_ref[...].astype(jnp.bfloat16).reshape(bb * nw, dsl)

    s = jnp.dot(xb, ws_ref[...], preferred_element_type=jnp.float32)
    sb = s.astype(jnp.bfloat16)

    sv = jnp.dot(sb, wv_ref[...], preferred_element_type=jnp.float32)

    t = jnp.dot(sb, qkt_ref[...], preferred_element_type=jnp.float32)

    ms = t.shape[-1]
    t3 = t.reshape(bb, nw, ms)

    m = jnp.maximum(jnp.max(t3, axis=1, keepdims=True), 0.0)
    p = jnp.exp(t3 - m)
    denom = jnp.sum(p, axis=1, keepdims=True) + jnp.exp(-m)
    p = (p / denom).astype(jnp.bfloat16)

    sv3 = sv.astype(jnp.bfloat16).reshape(bb, nw, md)
    mem = lax.dot_general(p, sv3, (((1,), (1,)), ((0,), (0,))),
                          preferred_element_type=jnp.float32)

    ms1 = jnp.mean(mem * mem, axis=(1, 2), keepdims=True)
    hn = h_ref[...] + mem * lax.rsqrt(ms1 + eps) * gmem_ref[...]
    ms2 = jnp.mean(hn * hn, axis=(1, 2), keepdims=True)
    o_ref[...] = (hn * lax.rsqrt(ms2 + eps) * gh_ref[...]).astype(o_ref.dtype)


def kernel(x, h, q, ws_w, wk_w, wv_w, g_mem, g_h):
    B, seqlen, dim = x.shape
    _, memory_size, memory_dim = h.shape
    summary_len = ws_w.shape[1] // dim
    nw = seqlen // summary_len
    dsl = dim * summary_len
    eps = float(jnp.finfo(jnp.float32).eps)
    scale = 1.0 / math.sqrt(memory_dim)

    ws_t = ws_w.T.astype(jnp.bfloat16)
    wv_t = wv_w.T.astype(jnp.bfloat16)
    qkt = ((q @ wk_w) * scale).T.astype(jnp.bfloat16)

    bb = _BB
    return pl.pallas_call(
        lambda *refs: _fused_kernel(*refs, eps=eps),
        out_shape=jax.ShapeDtypeStruct((B, memory_size, memory_dim), h.dtype),
        grid=(B // bb,),
        in_specs=[
            pl.BlockSpec((bb, seqlen, dim), lambda i: (i, 0, 0)),
            pl.BlockSpec((bb, memory_size, memory_dim), lambda i: (i, 0, 0)),
            pl.BlockSpec((memory_dim, memory_size), lambda i: (0, 0)),
            pl.BlockSpec((dsl, memory_dim), lambda i: (0, 0)),
            pl.BlockSpec((memory_dim, memory_dim), lambda i: (0, 0)),
            pl.BlockSpec((memory_size, memory_dim), lambda i: (0, 0)),
            pl.BlockSpec((memory_size, memory_dim), lambda i: (0, 0)),
        ],
        out_specs=pl.BlockSpec((bb, memory_size, memory_dim),
                               lambda i: (i, 0, 0)),
        compiler_params=pltpu.CompilerParams(
            dimension_semantics=("parallel",),
            vmem_limit_bytes=64 * 1024 * 1024,
        ),
    )(x, h, qkt, ws_t, wv_t, g_mem, g_h)


# speedup vs baseline: 13.2534x; 1.0200x over previous
import math

import jax
import jax.numpy as jnp
from jax import lax
from jax.experimental import pallas as pl
from jax.experimental.pallas import tpu as pltpu

_BB = 32


def _fused_kernel(x_ref, h_ref, qkt_ref, ws_ref, wv_ref, gmem_ref, gh_ref,
                  o_ref, *, eps):
    bb, seqlen, dim = x_ref.shape
    md, dsl = ws_ref.shape
    nw = (seqlen * dim) // dsl

    xb = x_ref[...].astype(jnp.bfloat16).reshape(bb * nw, dsl)

    s = lax.dot_general(xb, ws_ref[...], (((1,), (1,)), ((), ())),
                        preferred_element_type=jnp.float32)
    sb = s.astype(jnp.bfloat16)

    sv = lax.dot_general(sb, wv_ref[...], (((1,), (1,)), ((), ())),
                         preferred_element_type=jnp.float32)

    t = jnp.dot(sb, qkt_ref[...], preferred_element_type=jnp.float32)

    ms = t.shape[-1]
    t3 = t.reshape(bb, nw, ms)

    m = jnp.maximum(jnp.max(t3, axis=1, keepdims=True), 0.0)
    p = jnp.exp(t3 - m)
    denom = jnp.sum(p, axis=1, keepdims=True) + jnp.exp(-m)
    p = (p / denom).astype(jnp.bfloat16)

    sv3 = sv.astype(jnp.bfloat16).reshape(bb, nw, md)
    mem = lax.dot_general(p, sv3, (((1,), (1,)), ((0,), (0,))),
                          preferred_element_type=jnp.float32)

    ms1 = jnp.mean(mem * mem, axis=(1, 2), keepdims=True)
    hn = h_ref[...] + mem * lax.rsqrt(ms1 + eps) * gmem_ref[...]
    ms2 = jnp.mean(hn * hn, axis=(1, 2), keepdims=True)
    o_ref[...] = (hn * lax.rsqrt(ms2 + eps) * gh_ref[...]).astype(o_ref.dtype)


def kernel(x, h, q, ws_w, wk_w, wv_w, g_mem, g_h):
    B, seqlen, dim = x.shape
    _, memory_size, memory_dim = h.shape
    summary_len = ws_w.shape[1] // dim
    nw = seqlen // summary_len
    dsl = dim * summary_len
    eps = float(jnp.finfo(jnp.float32).eps)
    scale = 1.0 / math.sqrt(memory_dim)

    ws_b = ws_w.astype(jnp.bfloat16)
    wv_b = wv_w.astype(jnp.bfloat16)
    qkt = (lax.dot_general(wk_w, q, (((0,), (1,)), ((), ())))
           * scale).astype(jnp.bfloat16)

    bb = _BB

    def xmap(i):
        return (i, 0, 0)

    def wmap(i):
        return (0, 0)

    return pl.pallas_call(
        lambda *refs: _fused_kernel(*refs, eps=eps),
        out_shape=jax.ShapeDtypeStruct((B, memory_size, memory_dim), h.dtype),
        grid=(B // bb,),
        in_specs=[
            pl.BlockSpec((bb, seqlen, dim), xmap),
            pl.BlockSpec((bb, memory_size, memory_dim), xmap),
            pl.BlockSpec((memory_dim, memory_size), wmap),
            pl.BlockSpec((memory_dim, dsl), wmap),
            pl.BlockSpec((memory_dim, memory_dim), wmap),
            pl.BlockSpec((memory_size, memory_dim), wmap),
            pl.BlockSpec((memory_size, memory_dim), wmap),
        ],
        out_specs=pl.BlockSpec((bb, memory_size, memory_dim), xmap),
        compiler_params=pltpu.CompilerParams(
            dimension_semantics=("parallel",),
            vmem_limit_bytes=50 * 1024 * 1024,
        ),
    )(x, h, qkt, ws_b, wv_b, g_mem, g_h)
